# Initial kernel scaffold; baseline (speedup 1.0000x reference)
import functools

import jax
import jax.numpy as jnp
from jax import lax
from jax.experimental import pallas as pl
from jax.experimental.pallas import tpu as pltpu
from jax.experimental.pallas import tpu_sc as plsc

_N = 10000
_E = 320000
_DIN = 128
_DH = 16
_DOUT = 2
_NC = 2
_NS = 16
_NW = _NC * _NS
_EW = _E // _NW
_CHUNK = 80
_NCH = _EW // _CHUNK
_NPAD = 10240
_RPT = _NPAD // _NS

_mesh = plsc.VectorSubcoreMesh(
    core_axis_name="c", subcore_axis_name="s", num_cores=_NC, num_subcores=_NS
)


def _seg_body(with_count, feat, srcI, dstI, *rest):
    if with_count:
        (agg_out, cnt_out, agg_sh, cnt_sh, sidx, didx, rows, ones, zrow, zc,
         gsem) = rest
    else:
        agg_out, agg_sh, sidx, didx, rows, zrow, gsem = rest
    cid = lax.axis_index("c")
    tid = lax.axis_index("s")
    wid = tid * _NC + cid

    z16 = jnp.zeros((16,), jnp.float32)

    def _zb(i, carry):
        zrow[i, :] = z16
        return carry

    lax.fori_loop(0, 128, _zb, 0)
    for j in range(_RPT // 128):
        pltpu.sync_copy(zrow, agg_sh.at[pl.ds(tid * _RPT + j * 128, 128)])
    if with_count:
        for i in range(8):
            zc[pl.ds(i * 16, 16)] = z16
        for i in range(_CHUNK // 16):
            ones[pl.ds(i * 16, 16)] = jnp.ones((16,), jnp.float32)
        for j in range(_RPT // 128):
            pltpu.sync_copy(zc, cnt_sh.at[pl.ds(tid * _RPT + j * 128, 128)])

    pltpu.sync_copy(srcI.at[wid], sidx)
    pltpu.sync_copy(dstI.at[wid], didx)

    plsc.subcore_barrier()

    def _chunk(g, carry):
        pltpu.async_copy(feat.at[sidx.at[g]], rows, gsem).wait()
        pltpu.sync_copy(rows, agg_sh.at[didx.at[g]], add=True)
        if with_count:
            pltpu.sync_copy(ones, cnt_sh.at[didx.at[g]], add=True)
        return carry

    lax.fori_loop(0, _NCH, _chunk, 0)

    plsc.subcore_barrier()

    pltpu.sync_copy(agg_sh.at[pl.ds(tid * _RPT, _RPT)],
                    agg_out.at[cid, pl.ds(tid * _RPT, _RPT)])
    if with_count:
        pltpu.sync_copy(cnt_sh.at[pl.ds(tid * _RPT, _RPT)],
                        cnt_out.at[cid, pl.ds(tid * _RPT, _RPT)])


_seg_cnt = pl.kernel(
    functools.partial(_seg_body, True),
    out_type=[
        jax.ShapeDtypeStruct((_NC, _NPAD, _DH), jnp.float32),
        jax.ShapeDtypeStruct((_NC, _NPAD), jnp.float32),
    ],
    mesh=_mesh,
    scratch_types=[
        pltpu.VMEM_SHARED((_NPAD, _DH), jnp.float32),
        pltpu.VMEM_SHARED((_NPAD,), jnp.float32),
        pltpu.VMEM((_NCH, _CHUNK), jnp.int32),
        pltpu.VMEM((_NCH, _CHUNK), jnp.int32),
        pltpu.VMEM((_CHUNK, _DH), jnp.float32),
        pltpu.VMEM((_CHUNK,), jnp.float32),
        pltpu.VMEM((128, _DH), jnp.float32),
        pltpu.VMEM((128,), jnp.float32),
        pltpu.SemaphoreType.DMA,
    ],
)

_seg = pl.kernel(
    functools.partial(_seg_body, False),
    out_type=jax.ShapeDtypeStruct((_NC, _NPAD, _DH), jnp.float32),
    mesh=_mesh,
    scratch_types=[
        pltpu.VMEM_SHARED((_NPAD, _DH), jnp.float32),
        pltpu.VMEM((_NCH, _CHUNK), jnp.int32),
        pltpu.VMEM((_NCH, _CHUNK), jnp.int32),
        pltpu.VMEM((_CHUNK, _DH), jnp.float32),
        pltpu.VMEM((128, _DH), jnp.float32),
        pltpu.SemaphoreType.DMA,
    ],
)

_BLK = 2000
_GRID = _N // _BLK


def _mm1_body(x_ref, w_ref, o_ref):
    o_ref[...] = lax.dot_general(
        x_ref[...], w_ref[...], (((1,), (1,)), ((), ())),
        preferred_element_type=jnp.float32)


_mm1 = pl.pallas_call(
    _mm1_body,
    grid=(_GRID,),
    in_specs=[
        pl.BlockSpec((_BLK, _DIN), lambda i: (i, 0)),
        pl.BlockSpec((2 * _DH, _DIN), lambda i: (0, 0)),
    ],
    out_specs=pl.BlockSpec((_BLK, 2 * _DH), lambda i: (i, 0)),
    out_shape=jax.ShapeDtypeStruct((_N, 2 * _DH), jnp.float32),
)


def _tc2_body(a0, a1, c0, c1, z1, b1, h_ref, inv_ref):
    inv = 1.0 / jnp.maximum(c0[...] + c1[...], 1.0)
    inv_ref[...] = inv
    h_ref[...] = jnp.maximum((a0[...] + a1[...]) * inv + z1[...] + b1[...], 0.0)


_tc2 = pl.pallas_call(
    _tc2_body,
    grid=(_GRID,),
    in_specs=[
        pl.BlockSpec((_BLK, _DH), lambda i: (i, 0)),
        pl.BlockSpec((_BLK, _DH), lambda i: (i, 0)),
        pl.BlockSpec((_BLK, 1), lambda i: (i, 0)),
        pl.BlockSpec((_BLK, 1), lambda i: (i, 0)),
        pl.BlockSpec((_BLK, _DH), lambda i: (i, 0)),
        pl.BlockSpec((1, _DH), lambda i: (0, 0)),
    ],
    out_specs=[
        pl.BlockSpec((_BLK, _DH), lambda i: (i, 0)),
        pl.BlockSpec((_BLK, 1), lambda i: (i, 0)),
    ],
    out_shape=[
        jax.ShapeDtypeStruct((_N, _DH), jnp.float32),
        jax.ShapeDtypeStruct((_N, 1), jnp.float32),
    ],
)


def _tc3_body(g0, g1, inv, h, wl, wr, b2, o_ref):
    m = (g0[...] + g1[...]) * inv[...]
    o_ref[...] = (
        lax.dot_general(m, wl[...], (((1,), (1,)), ((), ())),
                        preferred_element_type=jnp.float32)
        + lax.dot_general(h[...], wr[...], (((1,), (1,)), ((), ())),
                          preferred_element_type=jnp.float32)
        + b2[...])


_tc3 = pl.pallas_call(
    _tc3_body,
    grid=(_GRID,),
    in_specs=[
        pl.BlockSpec((_BLK, _DH), lambda i: (i, 0)),
        pl.BlockSpec((_BLK, _DH), lambda i: (i, 0)),
        pl.BlockSpec((_BLK, 1), lambda i: (i, 0)),
        pl.BlockSpec((_BLK, _DH), lambda i: (i, 0)),
        pl.BlockSpec((_DOUT, _DH), lambda i: (0, 0)),
        pl.BlockSpec((_DOUT, _DH), lambda i: (0, 0)),
        pl.BlockSpec((1, _DOUT), lambda i: (0, 0)),
    ],
    out_specs=pl.BlockSpec((_BLK, _DOUT), lambda i: (i, 0)),
    out_shape=jax.ShapeDtypeStruct((_N, _DOUT), jnp.float32),
)


def kernel(x, edge_index, W1_l, b1, W1_r, W2_l, b2, W2_r):
    src = edge_index[0].astype(jnp.int32).reshape(_NW, _NCH, _CHUNK)
    dst = edge_index[1].astype(jnp.int32).reshape(_NW, _NCH, _CHUNK)
    W1cat = jnp.concatenate([W1_l, W1_r], axis=0)
    y1z1 = _mm1(x, W1cat)
    y1 = y1z1[:, :_DH]
    z1 = y1z1[:, _DH:]
    aggp, cntp = _seg_cnt(y1, src, dst)
    h, inv = _tc2(aggp[0, :_N], aggp[1, :_N],
                  cntp[0, :_N, None], cntp[1, :_N, None],
                  z1, b1.reshape(1, _DH))
    gp = _seg(h, src, dst)
    out = _tc3(gp[0, :_N], gp[1, :_N], inv, h,
               W2_l, W2_r, b2.reshape(1, _DOUT))
    return out


# speedup vs baseline: 10.2419x; 10.2419x over previous
import functools

import jax
import jax.numpy as jnp
from jax import lax
from jax.experimental import pallas as pl
from jax.experimental.pallas import tpu as pltpu
from jax.experimental.pallas import tpu_sc as plsc

_N = 10000
_E = 320000
_DIN = 128
_DH = 16
_DOUT = 2
_NC = 2
_NS = 16
_NW = _NC * _NS
_EW = _E // _NW
_CHUNK = 80
_NCH = _EW // _CHUNK
_NPAD = 10240
_RPT = _NPAD // _NS

_mesh = plsc.VectorSubcoreMesh(
    core_axis_name="c", subcore_axis_name="s", num_cores=_NC, num_subcores=_NS
)


def _seg_body(with_count, feat, srcI, dstI, *rest):
    if with_count:
        (agg_out, cnt_out, agg_sh, cnt_sh, sidx, didx, rows, ones, zrow, zc,
         gsem) = rest
    else:
        agg_out, agg_sh, sidx, didx, rows, zrow, gsem = rest
    cid = lax.axis_index("c")
    tid = lax.axis_index("s")
    wid = tid * _NC + cid

    z16 = jnp.zeros((16,), jnp.float32)

    def _zb(i, carry):
        zrow[i, :] = z16
        return carry

    lax.fori_loop(0, 128, _zb, 0)
    for j in range(_RPT // 128):
        pltpu.sync_copy(zrow, agg_sh.at[pl.ds(tid * _RPT + j * 128, 128)])
    if with_count:
        for i in range(8):
            zc[pl.ds(i * 16, 16)] = z16
        for i in range(_CHUNK // 16):
            ones[pl.ds(i * 16, 16)] = jnp.ones((16,), jnp.float32)
        for j in range(_RPT // 128):
            pltpu.sync_copy(zc, cnt_sh.at[pl.ds(tid * _RPT + j * 128, 128)])

    pltpu.sync_copy(srcI.at[wid], sidx)
    pltpu.sync_copy(dstI.at[wid], didx)

    plsc.subcore_barrier()

    def _chunk(g, carry):
        pltpu.async_copy(feat.at[sidx.at[g]], rows, gsem).wait()
        pltpu.sync_copy(rows, agg_sh.at[didx.at[g]], add=True)
        if with_count:
            pltpu.sync_copy(ones, cnt_sh.at[didx.at[g]], add=True)
        return carry

    lax.fori_loop(0, _NCH, _chunk, 0)

    plsc.subcore_barrier()

    pltpu.sync_copy(agg_sh.at[pl.ds(tid * _RPT, _RPT)],
                    agg_out.at[cid, pl.ds(tid * _RPT, _RPT)])
    if with_count:
        pltpu.sync_copy(cnt_sh.at[pl.ds(tid * _RPT, _RPT)],
                        cnt_out.at[cid, pl.ds(tid * _RPT, _RPT)])


_sc_params = pltpu.CompilerParams(use_tc_tiling_on_sc=False)

_seg_cnt = pl.kernel(
    functools.partial(_seg_body, True),
    compiler_params=_sc_params,
    out_type=[
        jax.ShapeDtypeStruct((_NC, _NPAD, _DH), jnp.float32),
        jax.ShapeDtypeStruct((_NC, _NPAD), jnp.float32),
    ],
    mesh=_mesh,
    scratch_types=[
        pltpu.VMEM_SHARED((_NPAD, _DH), jnp.float32),
        pltpu.VMEM_SHARED((_NPAD,), jnp.float32),
        pltpu.VMEM((_NCH, _CHUNK), jnp.int32),
        pltpu.VMEM((_NCH, _CHUNK), jnp.int32),
        pltpu.VMEM((_CHUNK, _DH), jnp.float32),
        pltpu.VMEM((_CHUNK,), jnp.float32),
        pltpu.VMEM((128, _DH), jnp.float32),
        pltpu.VMEM((128,), jnp.float32),
        pltpu.SemaphoreType.DMA,
    ],
)

_seg = pl.kernel(
    functools.partial(_seg_body, False),
    compiler_params=_sc_params,
    out_type=jax.ShapeDtypeStruct((_NC, _NPAD, _DH), jnp.float32),
    mesh=_mesh,
    scratch_types=[
        pltpu.VMEM_SHARED((_NPAD, _DH), jnp.float32),
        pltpu.VMEM((_NCH, _CHUNK), jnp.int32),
        pltpu.VMEM((_NCH, _CHUNK), jnp.int32),
        pltpu.VMEM((_CHUNK, _DH), jnp.float32),
        pltpu.VMEM((128, _DH), jnp.float32),
        pltpu.SemaphoreType.DMA,
    ],
)

_BLK = 2000
_GRID = _N // _BLK


def _mm1_body(x_ref, w_ref, o_ref):
    o_ref[...] = lax.dot_general(
        x_ref[...], w_ref[...], (((1,), (1,)), ((), ())),
        preferred_element_type=jnp.float32)


_mm1 = pl.pallas_call(
    _mm1_body,
    grid=(_GRID,),
    in_specs=[
        pl.BlockSpec((_BLK, _DIN), lambda i: (i, 0)),
        pl.BlockSpec((2 * _DH, _DIN), lambda i: (0, 0)),
    ],
    out_specs=pl.BlockSpec((_BLK, 2 * _DH), lambda i: (i, 0)),
    out_shape=jax.ShapeDtypeStruct((_N, 2 * _DH), jnp.float32),
)


def _tc2_body(a0, a1, c0, c1, z1, b1, h_ref, inv_ref):
    inv = 1.0 / jnp.maximum(c0[...] + c1[...], 1.0)
    inv_ref[...] = inv
    h_ref[...] = jnp.maximum((a0[...] + a1[...]) * inv + z1[...] + b1[...], 0.0)


_tc2 = pl.pallas_call(
    _tc2_body,
    grid=(_GRID,),
    in_specs=[
        pl.BlockSpec((_BLK, _DH), lambda i: (i, 0)),
        pl.BlockSpec((_BLK, _DH), lambda i: (i, 0)),
        pl.BlockSpec((_BLK, 1), lambda i: (i, 0)),
        pl.BlockSpec((_BLK, 1), lambda i: (i, 0)),
        pl.BlockSpec((_BLK, _DH), lambda i: (i, 0)),
        pl.BlockSpec((1, _DH), lambda i: (0, 0)),
    ],
    out_specs=[
        pl.BlockSpec((_BLK, _DH), lambda i: (i, 0)),
        pl.BlockSpec((_BLK, 1), lambda i: (i, 0)),
    ],
    out_shape=[
        jax.ShapeDtypeStruct((_N, _DH), jnp.float32),
        jax.ShapeDtypeStruct((_N, 1), jnp.float32),
    ],
)


def _tc3_body(g0, g1, inv, h, wl, wr, b2, o_ref):
    m = (g0[...] + g1[...]) * inv[...]
    o_ref[...] = (
        lax.dot_general(m, wl[...], (((1,), (1,)), ((), ())),
                        preferred_element_type=jnp.float32)
        + lax.dot_general(h[...], wr[...], (((1,), (1,)), ((), ())),
                          preferred_element_type=jnp.float32)
        + b2[...])


_tc3 = pl.pallas_call(
    _tc3_body,
    grid=(_GRID,),
    in_specs=[
        pl.BlockSpec((_BLK, _DH), lambda i: (i, 0)),
        pl.BlockSpec((_BLK, _DH), lambda i: (i, 0)),
        pl.BlockSpec((_BLK, 1), lambda i: (i, 0)),
        pl.BlockSpec((_BLK, _DH), lambda i: (i, 0)),
        pl.BlockSpec((_DOUT, _DH), lambda i: (0, 0)),
        pl.BlockSpec((_DOUT, _DH), lambda i: (0, 0)),
        pl.BlockSpec((1, _DOUT), lambda i: (0, 0)),
    ],
    out_specs=pl.BlockSpec((_BLK, _DOUT), lambda i: (i, 0)),
    out_shape=jax.ShapeDtypeStruct((_N, _DOUT), jnp.float32),
)


def kernel(x, edge_index, W1_l, b1, W1_r, W2_l, b2, W2_r):
    src = edge_index[0].astype(jnp.int32).reshape(_NW, _NCH, _CHUNK)
    dst = edge_index[1].astype(jnp.int32).reshape(_NW, _NCH, _CHUNK)
    W1cat = jnp.concatenate([W1_l, W1_r], axis=0)
    y1z1 = _mm1(x, W1cat)
    y1 = y1z1[:, :_DH]
    z1 = y1z1[:, _DH:]
    aggp, cntp = _seg_cnt(y1, src, dst)
    h, inv = _tc2(aggp[0, :_N], aggp[1, :_N],
                  cntp[0, :_N, None], cntp[1, :_N, None],
                  z1, b1.reshape(1, _DH))
    gp = _seg(h, src, dst)
    out = _tc3(gp[0, :_N], gp[1, :_N], inv, h,
               W2_l, W2_r, b2.reshape(1, _DOUT))
    return out


# speedup vs baseline: 19.2005x; 1.8747x over previous
import functools

import jax
import jax.numpy as jnp
from jax import lax
from jax.experimental import pallas as pl
from jax.experimental.pallas import tpu as pltpu
from jax.experimental.pallas import tpu_sc as plsc

_N = 10000
_E = 320000
_DIN = 128
_DH = 16
_DOUT = 2
_NC = 2
_NS = 16
_NW = _NC * _NS
_EW = _E // _NW
_CHUNK = 80
_NCH = _EW // _CHUNK
_NPAD = 10240
_RPT = _NPAD // _NS

_mesh = plsc.VectorSubcoreMesh(
    core_axis_name="c", subcore_axis_name="s", num_cores=_NC, num_subcores=_NS
)


_NBUF = 5
_NGRP = _NCH // _NBUF


def _seg_body(with_count, feat, srcI, dstI, *rest):
    if with_count:
        (agg_out, cnt_out, agg_sh, cnt_sh, sidx, didx, rows, ones, zrow, zc,
         *sems) = rest
        gsem = sems[0:_NBUF]
        ssem = sems[_NBUF:2 * _NBUF]
        csem = sems[2 * _NBUF:3 * _NBUF]
    else:
        agg_out, agg_sh, sidx, didx, rows, zrow, *sems = rest
        gsem = sems[0:_NBUF]
        ssem = sems[_NBUF:2 * _NBUF]
        csem = None
    cid = lax.axis_index("c")
    tid = lax.axis_index("s")
    wid = tid * _NC + cid

    z16 = jnp.zeros((16,), jnp.float32)

    def _zb(i, carry):
        zrow[i, :] = z16
        return carry

    lax.fori_loop(0, 128, _zb, 0)
    for j in range(_RPT // 128):
        pltpu.sync_copy(zrow, agg_sh.at[pl.ds(tid * _RPT + j * 128, 128)])
    if with_count:
        for i in range(8):
            zc[pl.ds(i * 16, 16)] = z16
        for i in range(_CHUNK // 16):
            ones[pl.ds(i * 16, 16)] = jnp.ones((16,), jnp.float32)
        for j in range(_RPT // 128):
            pltpu.sync_copy(zc, cnt_sh.at[pl.ds(tid * _RPT + j * 128, 128)])

    pltpu.sync_copy(srcI.at[wid], sidx)
    pltpu.sync_copy(dstI.at[wid], didx)

    plsc.subcore_barrier()

    def _drain_b(b):
        pltpu.make_async_copy(rows.at[b], agg_sh.at[didx.at[0]],
                              ssem[b]).wait()
        if with_count:
            pltpu.make_async_copy(ones, cnt_sh.at[didx.at[0]],
                                  csem[b]).wait()

    def _group(gi, carry):
        base = gi * _NBUF
        for b in range(_NBUF):
            @pl.when(gi > 0)
            def _():
                _drain_b(b)

            pltpu.async_copy(feat.at[sidx.at[base + b]], rows.at[b], gsem[b])
        for b in range(_NBUF):
            pltpu.make_async_copy(feat.at[sidx.at[0]], rows.at[b],
                                  gsem[b]).wait()
            pltpu.async_copy(rows.at[b], agg_sh.at[didx.at[base + b]],
                             ssem[b], add=True)
            if with_count:
                pltpu.async_copy(ones, cnt_sh.at[didx.at[base + b]],
                                 csem[b], add=True)
        return carry

    lax.fori_loop(0, _NGRP, _group, 0)
    for b in range(_NBUF):
        _drain_b(b)

    plsc.subcore_barrier()

    pltpu.sync_copy(agg_sh.at[pl.ds(tid * _RPT, _RPT)],
                    agg_out.at[cid, pl.ds(tid * _RPT, _RPT)])
    if with_count:
        pltpu.sync_copy(cnt_sh.at[pl.ds(tid * _RPT, _RPT)],
                        cnt_out.at[cid, pl.ds(tid * _RPT, _RPT)])


_sc_params = pltpu.CompilerParams(use_tc_tiling_on_sc=False)

_seg_cnt = pl.kernel(
    functools.partial(_seg_body, True),
    compiler_params=_sc_params,
    out_type=[
        jax.ShapeDtypeStruct((_NC, _NPAD, _DH), jnp.float32),
        jax.ShapeDtypeStruct((_NC, _NPAD), jnp.float32),
    ],
    mesh=_mesh,
    scratch_types=[
        pltpu.VMEM_SHARED((_NPAD, _DH), jnp.float32),
        pltpu.VMEM_SHARED((_NPAD,), jnp.float32),
        pltpu.VMEM((_NCH, _CHUNK), jnp.int32),
        pltpu.VMEM((_NCH, _CHUNK), jnp.int32),
        pltpu.VMEM((_NBUF, _CHUNK, _DH), jnp.float32),
        pltpu.VMEM((_CHUNK,), jnp.float32),
        pltpu.VMEM((128, _DH), jnp.float32),
        pltpu.VMEM((128,), jnp.float32),
    ] + [pltpu.SemaphoreType.DMA] * (3 * _NBUF),
)

_seg = pl.kernel(
    functools.partial(_seg_body, False),
    compiler_params=_sc_params,
    out_type=jax.ShapeDtypeStruct((_NC, _NPAD, _DH), jnp.float32),
    mesh=_mesh,
    scratch_types=[
        pltpu.VMEM_SHARED((_NPAD, _DH), jnp.float32),
        pltpu.VMEM((_NCH, _CHUNK), jnp.int32),
        pltpu.VMEM((_NCH, _CHUNK), jnp.int32),
        pltpu.VMEM((_NBUF, _CHUNK, _DH), jnp.float32),
        pltpu.VMEM((128, _DH), jnp.float32),
    ] + [pltpu.SemaphoreType.DMA] * (2 * _NBUF),
)

_BLK = 2000
_GRID = _N // _BLK


def _mm1_body(x_ref, w_ref, o_ref):
    o_ref[...] = lax.dot_general(
        x_ref[...], w_ref[...], (((1,), (1,)), ((), ())),
        preferred_element_type=jnp.float32)


_mm1 = pl.pallas_call(
    _mm1_body,
    grid=(_GRID,),
    in_specs=[
        pl.BlockSpec((_BLK, _DIN), lambda i: (i, 0)),
        pl.BlockSpec((2 * _DH, _DIN), lambda i: (0, 0)),
    ],
    out_specs=pl.BlockSpec((_BLK, 2 * _DH), lambda i: (i, 0)),
    out_shape=jax.ShapeDtypeStruct((_N, 2 * _DH), jnp.float32),
)


def _tc2_body(a0, a1, c0, c1, z1, b1, h_ref, inv_ref):
    inv = 1.0 / jnp.maximum(c0[...] + c1[...], 1.0)
    inv_ref[...] = inv
    h_ref[...] = jnp.maximum((a0[...] + a1[...]) * inv + z1[...] + b1[...], 0.0)


_tc2 = pl.pallas_call(
    _tc2_body,
    grid=(_GRID,),
    in_specs=[
        pl.BlockSpec((_BLK, _DH), lambda i: (i, 0)),
        pl.BlockSpec((_BLK, _DH), lambda i: (i, 0)),
        pl.BlockSpec((_BLK, 1), lambda i: (i, 0)),
        pl.BlockSpec((_BLK, 1), lambda i: (i, 0)),
        pl.BlockSpec((_BLK, _DH), lambda i: (i, 0)),
        pl.BlockSpec((1, _DH), lambda i: (0, 0)),
    ],
    out_specs=[
        pl.BlockSpec((_BLK, _DH), lambda i: (i, 0)),
        pl.BlockSpec((_BLK, 1), lambda i: (i, 0)),
    ],
    out_shape=[
        jax.ShapeDtypeStruct((_N, _DH), jnp.float32),
        jax.ShapeDtypeStruct((_N, 1), jnp.float32),
    ],
)


def _tc3_body(g0, g1, inv, h, wl, wr, b2, o_ref):
    m = (g0[...] + g1[...]) * inv[...]
    o_ref[...] = (
        lax.dot_general(m, wl[...], (((1,), (1,)), ((), ())),
                        preferred_element_type=jnp.float32)
        + lax.dot_general(h[...], wr[...], (((1,), (1,)), ((), ())),
                          preferred_element_type=jnp.float32)
        + b2[...])


_tc3 = pl.pallas_call(
    _tc3_body,
    grid=(_GRID,),
    in_specs=[
        pl.BlockSpec((_BLK, _DH), lambda i: (i, 0)),
        pl.BlockSpec((_BLK, _DH), lambda i: (i, 0)),
        pl.BlockSpec((_BLK, 1), lambda i: (i, 0)),
        pl.BlockSpec((_BLK, _DH), lambda i: (i, 0)),
        pl.BlockSpec((_DOUT, _DH), lambda i: (0, 0)),
        pl.BlockSpec((_DOUT, _DH), lambda i: (0, 0)),
        pl.BlockSpec((1, _DOUT), lambda i: (0, 0)),
    ],
    out_specs=pl.BlockSpec((_BLK, _DOUT), lambda i: (i, 0)),
    out_shape=jax.ShapeDtypeStruct((_N, _DOUT), jnp.float32),
)


def kernel(x, edge_index, W1_l, b1, W1_r, W2_l, b2, W2_r):
    src = edge_index[0].astype(jnp.int32).reshape(_NW, _NCH, _CHUNK)
    dst = edge_index[1].astype(jnp.int32).reshape(_NW, _NCH, _CHUNK)
    W1cat = jnp.concatenate([W1_l, W1_r], axis=0)
    y1z1 = _mm1(x, W1cat)
    y1 = y1z1[:, :_DH]
    z1 = y1z1[:, _DH:]
    aggp, cntp = _seg_cnt(y1, src, dst)
    h, inv = _tc2(aggp[0, :_N], aggp[1, :_N],
                  cntp[0, :_N, None], cntp[1, :_N, None],
                  z1, b1.reshape(1, _DH))
    gp = _seg(h, src, dst)
    out = _tc3(gp[0, :_N], gp[1, :_N], inv, h,
               W2_l, W2_r, b2.reshape(1, _DOUT))
    return out


# speedup vs baseline: 21.7350x; 1.1320x over previous
import functools

import jax
import jax.numpy as jnp
from jax import lax
from jax.experimental import pallas as pl
from jax.experimental.pallas import tpu as pltpu
from jax.experimental.pallas import tpu_sc as plsc

_N = 10000
_E = 320000
_DIN = 128
_DH = 16
_DOUT = 2
_NC = 2
_NS = 16
_NW = _NC * _NS
_EW = _E // _NW
_CHUNK = 400
_NCH = _EW // _CHUNK
_NPAD = 10240
_RPT = _NPAD // _NS

_mesh = plsc.VectorSubcoreMesh(
    core_axis_name="c", subcore_axis_name="s", num_cores=_NC, num_subcores=_NS
)


_NBUF = 5
_NGRP = _NCH // _NBUF


def _seg_body(with_count, feat, srcI, dstI, *rest):
    if with_count:
        (agg_out, cnt_out, agg_sh, cnt_sh, sidx, didx, rows, ones, zrow, zc,
         *sems) = rest
        gsem = sems[0:_NBUF]
        ssem = sems[_NBUF:2 * _NBUF]
        csem = sems[2 * _NBUF:3 * _NBUF]
    else:
        agg_out, agg_sh, sidx, didx, rows, zrow, *sems = rest
        gsem = sems[0:_NBUF]
        ssem = sems[_NBUF:2 * _NBUF]
        csem = None
    cid = lax.axis_index("c")
    tid = lax.axis_index("s")
    wid = tid * _NC + cid

    z16 = jnp.zeros((16,), jnp.float32)

    def _zb(i, carry):
        zrow[i, :] = z16
        return carry

    lax.fori_loop(0, 128, _zb, 0)
    for j in range(_RPT // 128):
        pltpu.sync_copy(zrow, agg_sh.at[pl.ds(tid * _RPT + j * 128, 128)])
    if with_count:
        for i in range(8):
            zc[pl.ds(i * 16, 16)] = z16
        for i in range(_CHUNK // 16):
            ones[pl.ds(i * 16, 16)] = jnp.ones((16,), jnp.float32)
        for j in range(_RPT // 128):
            pltpu.sync_copy(zc, cnt_sh.at[pl.ds(tid * _RPT + j * 128, 128)])

    pltpu.sync_copy(srcI.at[wid], sidx)
    pltpu.sync_copy(dstI.at[wid], didx)

    plsc.subcore_barrier()

    def _drain_b(b):
        pltpu.make_async_copy(rows.at[b], agg_sh.at[didx.at[0]],
                              ssem[b]).wait()
        if with_count:
            pltpu.make_async_copy(ones, cnt_sh.at[didx.at[0]],
                                  csem[b]).wait()

    def _group(gi, carry):
        base = gi * _NBUF
        for b in range(_NBUF):
            @pl.when(gi > 0)
            def _():
                _drain_b(b)

            pltpu.async_copy(feat.at[sidx.at[base + b]], rows.at[b], gsem[b])
        for b in range(_NBUF):
            pltpu.make_async_copy(feat.at[sidx.at[0]], rows.at[b],
                                  gsem[b]).wait()
            pltpu.async_copy(rows.at[b], agg_sh.at[didx.at[base + b]],
                             ssem[b], add=True)
            if with_count:
                pltpu.async_copy(ones, cnt_sh.at[didx.at[base + b]],
                                 csem[b], add=True)
        return carry

    lax.fori_loop(0, _NGRP, _group, 0)
    for b in range(_NBUF):
        _drain_b(b)

    plsc.subcore_barrier()

    pltpu.sync_copy(agg_sh.at[pl.ds(tid * _RPT, _RPT)],
                    agg_out.at[cid, pl.ds(tid * _RPT, _RPT)])
    if with_count:
        pltpu.sync_copy(cnt_sh.at[pl.ds(tid * _RPT, _RPT)],
                        cnt_out.at[cid, pl.ds(tid * _RPT, _RPT)])


_sc_params = pltpu.CompilerParams(use_tc_tiling_on_sc=False)

_seg_cnt = pl.kernel(
    functools.partial(_seg_body, True),
    compiler_params=_sc_params,
    out_type=[
        jax.ShapeDtypeStruct((_NC, _NPAD, _DH), jnp.float32),
        jax.ShapeDtypeStruct((_NC, _NPAD), jnp.float32),
    ],
    mesh=_mesh,
    scratch_types=[
        pltpu.VMEM_SHARED((_NPAD, _DH), jnp.float32),
        pltpu.VMEM_SHARED((_NPAD,), jnp.float32),
        pltpu.VMEM((_NCH, _CHUNK), jnp.int32),
        pltpu.VMEM((_NCH, _CHUNK), jnp.int32),
        pltpu.VMEM((_NBUF, _CHUNK, _DH), jnp.float32),
        pltpu.VMEM((_CHUNK,), jnp.float32),
        pltpu.VMEM((128, _DH), jnp.float32),
        pltpu.VMEM((128,), jnp.float32),
    ] + [pltpu.SemaphoreType.DMA] * (3 * _NBUF),
)

_seg = pl.kernel(
    functools.partial(_seg_body, False),
    compiler_params=_sc_params,
    out_type=jax.ShapeDtypeStruct((_NC, _NPAD, _DH), jnp.float32),
    mesh=_mesh,
    scratch_types=[
        pltpu.VMEM_SHARED((_NPAD, _DH), jnp.float32),
        pltpu.VMEM((_NCH, _CHUNK), jnp.int32),
        pltpu.VMEM((_NCH, _CHUNK), jnp.int32),
        pltpu.VMEM((_NBUF, _CHUNK, _DH), jnp.float32),
        pltpu.VMEM((128, _DH), jnp.float32),
    ] + [pltpu.SemaphoreType.DMA] * (2 * _NBUF),
)

_BLK = 2000
_GRID = _N // _BLK


def _mm1_body(x_ref, w_ref, o_ref):
    o_ref[...] = lax.dot_general(
        x_ref[...], w_ref[...], (((1,), (1,)), ((), ())),
        preferred_element_type=jnp.float32)


_mm1 = pl.pallas_call(
    _mm1_body,
    grid=(_GRID,),
    in_specs=[
        pl.BlockSpec((_BLK, _DIN), lambda i: (i, 0)),
        pl.BlockSpec((2 * _DH, _DIN), lambda i: (0, 0)),
    ],
    out_specs=pl.BlockSpec((_BLK, 2 * _DH), lambda i: (i, 0)),
    out_shape=jax.ShapeDtypeStruct((_N, 2 * _DH), jnp.float32),
)


def _tc2_body(a0, a1, c0, c1, z1, b1, h_ref, inv_ref):
    inv = 1.0 / jnp.maximum(c0[...] + c1[...], 1.0)
    inv_ref[...] = inv
    h_ref[...] = jnp.maximum((a0[...] + a1[...]) * inv + z1[...] + b1[...], 0.0)


_tc2 = pl.pallas_call(
    _tc2_body,
    grid=(_GRID,),
    in_specs=[
        pl.BlockSpec((_BLK, _DH), lambda i: (i, 0)),
        pl.BlockSpec((_BLK, _DH), lambda i: (i, 0)),
        pl.BlockSpec((_BLK, 1), lambda i: (i, 0)),
        pl.BlockSpec((_BLK, 1), lambda i: (i, 0)),
        pl.BlockSpec((_BLK, _DH), lambda i: (i, 0)),
        pl.BlockSpec((1, _DH), lambda i: (0, 0)),
    ],
    out_specs=[
        pl.BlockSpec((_BLK, _DH), lambda i: (i, 0)),
        pl.BlockSpec((_BLK, 1), lambda i: (i, 0)),
    ],
    out_shape=[
        jax.ShapeDtypeStruct((_N, _DH), jnp.float32),
        jax.ShapeDtypeStruct((_N, 1), jnp.float32),
    ],
)


def _tc3_body(g0, g1, inv, h, wl, wr, b2, o_ref):
    m = (g0[...] + g1[...]) * inv[...]
    o_ref[...] = (
        lax.dot_general(m, wl[...], (((1,), (1,)), ((), ())),
                        preferred_element_type=jnp.float32)
        + lax.dot_general(h[...], wr[...], (((1,), (1,)), ((), ())),
                          preferred_element_type=jnp.float32)
        + b2[...])


_tc3 = pl.pallas_call(
    _tc3_body,
    grid=(_GRID,),
    in_specs=[
        pl.BlockSpec((_BLK, _DH), lambda i: (i, 0)),
        pl.BlockSpec((_BLK, _DH), lambda i: (i, 0)),
        pl.BlockSpec((_BLK, 1), lambda i: (i, 0)),
        pl.BlockSpec((_BLK, _DH), lambda i: (i, 0)),
        pl.BlockSpec((_DOUT, _DH), lambda i: (0, 0)),
        pl.BlockSpec((_DOUT, _DH), lambda i: (0, 0)),
        pl.BlockSpec((1, _DOUT), lambda i: (0, 0)),
    ],
    out_specs=pl.BlockSpec((_BLK, _DOUT), lambda i: (i, 0)),
    out_shape=jax.ShapeDtypeStruct((_N, _DOUT), jnp.float32),
)


def kernel(x, edge_index, W1_l, b1, W1_r, W2_l, b2, W2_r):
    src = edge_index[0].astype(jnp.int32).reshape(_NW, _NCH, _CHUNK)
    dst = edge_index[1].astype(jnp.int32).reshape(_NW, _NCH, _CHUNK)
    W1cat = jnp.concatenate([W1_l, W1_r], axis=0)
    y1z1 = _mm1(x, W1cat)
    y1 = y1z1[:, :_DH]
    z1 = y1z1[:, _DH:]
    aggp, cntp = _seg_cnt(y1, src, dst)
    h, inv = _tc2(aggp[0, :_N], aggp[1, :_N],
                  cntp[0, :_N, None], cntp[1, :_N, None],
                  z1, b1.reshape(1, _DH))
    gp = _seg(h, src, dst)
    out = _tc3(gp[0, :_N], gp[1, :_N], inv, h,
               W2_l, W2_r, b2.reshape(1, _DOUT))
    return out


# speedup vs baseline: 35.7943x; 1.6468x over previous
import functools

import jax
import jax.numpy as jnp
from jax import lax
from jax.experimental import pallas as pl
from jax.experimental.pallas import tpu as pltpu
from jax.experimental.pallas import tpu_sc as plsc

_N = 10000
_E = 320000
_DIN = 128
_DH = 16
_DOUT = 2
_NC = 2
_NS = 16
_NW = _NC * _NS
_EW = _E // _NW
_CHUNK = 400
_NCH = _EW // _CHUNK
_NPAD = 10240
_RPT = _NPAD // _NS
_PK = _NPAD * _DH // 128

_mesh = plsc.VectorSubcoreMesh(
    core_axis_name="c", subcore_axis_name="s", num_cores=_NC, num_subcores=_NS
)

_NBUF = 5
_NGRP = _NCH // _NBUF


def _seg_body(with_count, feat, edges, *rest):
    if with_count:
        (agg_out, cnt_out, agg_sh, cnt_sh, sidx, didx, rows, ones, zrow,
         *sems) = rest
        gsem = sems[0:_NBUF]
        ssem = sems[_NBUF:2 * _NBUF]
        csem = sems[2 * _NBUF:3 * _NBUF]
    else:
        agg_out, agg_sh, sidx, didx, rows, zrow, *sems = rest
        gsem = sems[0:_NBUF]
        ssem = sems[_NBUF:2 * _NBUF]
        csem = None
    cid = lax.axis_index("c")
    tid = lax.axis_index("s")
    wid = tid * _NC + cid

    z16 = jnp.zeros((16,), jnp.float32)

    def _zb(i, carry):
        zrow[i, :] = z16
        return carry

    lax.fori_loop(0, 128, _zb, 0)
    for j in range(_RPT // 128):
        pltpu.sync_copy(zrow, agg_sh.at[pl.ds(tid * _RPT + j * 128, 128)])
    if with_count:
        o16 = jnp.ones((16,), jnp.float32)

        def _ob(i, carry):
            ones[i, :] = o16
            return carry

        lax.fori_loop(0, _CHUNK, _ob, 0)
        for j in range(_RPT // 128):
            pltpu.sync_copy(zrow, cnt_sh.at[pl.ds(tid * _RPT + j * 128, 128)])

    pltpu.sync_copy(edges.at[0, wid], sidx)
    pltpu.sync_copy(edges.at[1, wid], didx)

    plsc.subcore_barrier()

    def _dix(g):
        return didx.at[g]

    def _six(g):
        return sidx.at[g]

    def _drain_b(b):
        pltpu.make_async_copy(rows.at[b], agg_sh.at[_dix(0)], ssem[b]).wait()
        if with_count:
            pltpu.make_async_copy(ones, cnt_sh.at[_dix(0)], csem[b]).wait()

    def _group(gi, carry):
        base = gi * _NBUF
        for b in range(_NBUF):
            @pl.when(gi > 0)
            def _():
                _drain_b(b)

            pltpu.async_copy(feat.at[_six(base + b)], rows.at[b], gsem[b])
        for b in range(_NBUF):
            pltpu.make_async_copy(feat.at[_six(0)], rows.at[b],
                                  gsem[b]).wait()
            pltpu.async_copy(rows.at[b], agg_sh.at[_dix(base + b)],
                             ssem[b], add=True)
            if with_count:
                pltpu.async_copy(ones, cnt_sh.at[_dix(base + b)],
                                 csem[b], add=True)
        return carry

    lax.fori_loop(0, _NGRP, _group, 0)
    for b in range(_NBUF):
        _drain_b(b)

    plsc.subcore_barrier()

    pltpu.sync_copy(agg_sh.at[pl.ds(tid * _RPT, _RPT)],
                    agg_out.at[cid, pl.ds(tid * _RPT, _RPT)])
    if with_count:
        pltpu.sync_copy(cnt_sh.at[pl.ds(tid * _RPT, _RPT)],
                        cnt_out.at[cid, pl.ds(tid * _RPT, _RPT)])


_sc_params = pltpu.CompilerParams(use_tc_tiling_on_sc=False)

_seg_cnt = pl.kernel(
    functools.partial(_seg_body, True),
    compiler_params=_sc_params,
    out_type=[
        jax.ShapeDtypeStruct((_NC, _NPAD, _DH), jnp.float32),
        jax.ShapeDtypeStruct((_NC, _NPAD, _DH), jnp.float32),
    ],
    mesh=_mesh,
    scratch_types=[
        pltpu.VMEM_SHARED((_NPAD, _DH), jnp.float32),
        pltpu.VMEM_SHARED((_NPAD, _DH), jnp.float32),
        pltpu.VMEM((_NCH, _CHUNK), jnp.int32),
        pltpu.VMEM((_NCH, _CHUNK), jnp.int32),
        pltpu.VMEM((_NBUF, _CHUNK, _DH), jnp.float32),
        pltpu.VMEM((_CHUNK, _DH), jnp.float32),
        pltpu.VMEM((128, _DH), jnp.float32),
    ] + [pltpu.SemaphoreType.DMA] * (3 * _NBUF),
)

_seg = pl.kernel(
    functools.partial(_seg_body, False),
    compiler_params=_sc_params,
    out_type=jax.ShapeDtypeStruct((_NC, _NPAD, _DH), jnp.float32),
    mesh=_mesh,
    scratch_types=[
        pltpu.VMEM_SHARED((_NPAD, _DH), jnp.float32),
        pltpu.VMEM((_NCH, _CHUNK), jnp.int32),
        pltpu.VMEM((_NCH, _CHUNK), jnp.int32),
        pltpu.VMEM((_NBUF, _CHUNK, _DH), jnp.float32),
        pltpu.VMEM((128, _DH), jnp.float32),
    ] + [pltpu.SemaphoreType.DMA] * (2 * _NBUF),
)

_BLKN = 2048
_BLKP = _BLKN * _DH // 128
_GRID = _NPAD // _BLKN


def _mm1_body(x_ref, w_ref, y_ref, z_ref, ty_ref, tz_ref):
    t = lax.dot_general(
        x_ref[...], w_ref[...], (((1,), (1,)), ((), ())),
        preferred_element_type=jnp.float32)
    ty_ref[...] = t[:, 0:_DH]
    tz_ref[...] = t[:, _DH:2 * _DH]
    for n8 in range(8):
        y_ref[:, n8 * _DH:(n8 + 1) * _DH] = ty_ref[n8::8, :]
        z_ref[:, n8 * _DH:(n8 + 1) * _DH] = tz_ref[n8::8, :]


_mm1 = pl.pallas_call(
    _mm1_body,
    grid=(_GRID,),
    in_specs=[
        pl.BlockSpec((_BLKN, _DIN), lambda i: (i, 0)),
        pl.BlockSpec((2 * _DH, _DIN), lambda i: (0, 0)),
    ],
    out_specs=[
        pl.BlockSpec((_BLKP, 128), lambda i: (i, 0)),
        pl.BlockSpec((_BLKP, 128), lambda i: (i, 0)),
    ],
    out_shape=[
        jax.ShapeDtypeStruct((_PK, 128), jnp.float32),
        jax.ShapeDtypeStruct((_PK, 128), jnp.float32),
    ],
    scratch_shapes=[pltpu.VMEM((_BLKN, _DH), jnp.float32),
                    pltpu.VMEM((_BLKN, _DH), jnp.float32)],
)


def _tc2_body(ap, cp, zp, b1p, h_ref, inv_ref):
    a = ap[0] + ap[1]
    c = cp[0] + cp[1]
    inv = 1.0 / jnp.maximum(c, 1.0)
    inv_ref[...] = inv
    h_ref[...] = jnp.maximum(a * inv + zp[...] + b1p[...], 0.0)


_tc2 = pl.pallas_call(
    _tc2_body,
    grid=(_GRID,),
    in_specs=[
        pl.BlockSpec((_NC, _BLKP, 128), lambda i: (0, i, 0)),
        pl.BlockSpec((_NC, _BLKP, 128), lambda i: (0, i, 0)),
        pl.BlockSpec((_BLKP, 128), lambda i: (i, 0)),
        pl.BlockSpec((1, 128), lambda i: (0, 0)),
    ],
    out_specs=[
        pl.BlockSpec((_BLKP, 128), lambda i: (i, 0)),
        pl.BlockSpec((_BLKP, 128), lambda i: (i, 0)),
    ],
    out_shape=[
        jax.ShapeDtypeStruct((_PK, 128), jnp.float32),
        jax.ShapeDtypeStruct((_PK, 128), jnp.float32),
    ],
)


def _tc3_body(gp, invp, hp, bm, bh, b2t, o_ref):
    m = (gp[0] + gp[1]) * invp[...]
    o_ref[...] = (
        lax.dot_general(m, bm[...], (((1,), (0,)), ((), ())),
                        preferred_element_type=jnp.float32)
        + lax.dot_general(hp[...], bh[...], (((1,), (0,)), ((), ())),
                          preferred_element_type=jnp.float32)
        + b2t[...])


_tc3 = pl.pallas_call(
    _tc3_body,
    grid=(_GRID,),
    in_specs=[
        pl.BlockSpec((_NC, _BLKP, 128), lambda i: (0, i, 0)),
        pl.BlockSpec((_BLKP, 128), lambda i: (i, 0)),
        pl.BlockSpec((_BLKP, 128), lambda i: (i, 0)),
        pl.BlockSpec((8 * _DH, 8 * _DOUT), lambda i: (0, 0)),
        pl.BlockSpec((8 * _DH, 8 * _DOUT), lambda i: (0, 0)),
        pl.BlockSpec((1, 8 * _DOUT), lambda i: (0, 0)),
    ],
    out_specs=pl.BlockSpec((_BLKP, 8 * _DOUT), lambda i: (i, 0)),
    out_shape=jax.ShapeDtypeStruct((_PK, 8 * _DOUT), jnp.float32),
)


def _expand_w2(w):
    eye8 = jnp.eye(8, dtype=w.dtype)
    return jnp.einsum("ab,fo->afbo", eye8, w.T).reshape(8 * _DH, 8 * _DOUT)


def kernel(x, edge_index, W1_l, b1, W1_r, W2_l, b2, W2_r):
    edges = edge_index.astype(jnp.int32).reshape(2, _NW, _NCH, _CHUNK)
    W1cat = jnp.concatenate([W1_l, W1_r], axis=0)
    y1p, z1p = _mm1(x, W1cat)
    aggp, cntp = _seg_cnt(y1p.reshape(_NPAD, _DH), edges)
    hp, invp = _tc2(aggp.reshape(_NC, _PK, 128),
                    cntp.reshape(_NC, _PK, 128),
                    z1p, jnp.tile(b1, 8).reshape(1, 128))
    gp = _seg(hp.reshape(_NPAD, _DH), edges)
    op = _tc3(gp.reshape(_NC, _PK, 128), invp, hp,
              _expand_w2(W2_l), _expand_w2(W2_r),
              jnp.tile(b2, 8).reshape(1, 8 * _DOUT))
    return op.reshape(_NPAD, _DOUT)[:_N]


# speedup vs baseline: 36.5758x; 1.0218x over previous
import functools

import jax
import jax.numpy as jnp
from jax import lax
from jax.experimental import pallas as pl
from jax.experimental.pallas import tpu as pltpu
from jax.experimental.pallas import tpu_sc as plsc

_N = 10000
_E = 320000
_DIN = 128
_DH = 16
_DOUT = 2
_NC = 2
_NS = 16
_NW = _NC * _NS
_EW = _E // _NW
_CHUNK = 400
_NCH = _EW // _CHUNK
_NPAD = 10240
_RPT = _NPAD // _NS
_PK = _NPAD * _DH // 128

_mesh = plsc.VectorSubcoreMesh(
    core_axis_name="c", subcore_axis_name="s", num_cores=_NC, num_subcores=_NS
)

_NBUF = 5
_NGRP = _NCH // _NBUF


def _seg_body(with_count, feat, edges, *rest):
    if with_count:
        (agg_out, cnt_out, agg_sh, cnt_sh, sidx, didx, rows, ones, zrow,
         zrow16, *sems) = rest
        gsem = sems[0:_NBUF]
        ssem = sems[_NBUF:2 * _NBUF]
        csem = sems[2 * _NBUF:3 * _NBUF]
    else:
        agg_out, agg_sh, sidx, didx, rows, zrow, *sems = rest
        gsem = sems[0:_NBUF]
        ssem = sems[_NBUF:2 * _NBUF]
        csem = None
    cid = lax.axis_index("c")
    tid = lax.axis_index("s")
    wid = tid * _NC + cid

    z16 = jnp.zeros((16,), jnp.float32)

    def _zb(i, carry):
        zrow[i, :] = z16
        return carry

    lax.fori_loop(0, 128, _zb, 0)
    for j in range(_RPT // 128):
        pltpu.sync_copy(zrow, agg_sh.at[pl.ds(tid * _RPT + j * 128, 128)])
    if with_count:
        o2 = jnp.ones((2, 16), jnp.int16)
        zs2 = jnp.zeros((2, 16), jnp.int16)

        def _ob(i, carry):
            ones[pl.ds(i * 2, 2), :] = o2
            return carry

        lax.fori_loop(0, _CHUNK // 2, _ob, 0)

        def _zsb(i, carry):
            zrow16[pl.ds(i * 2, 2), :] = zs2
            return carry

        lax.fori_loop(0, 64, _zsb, 0)
        for j in range(_RPT // 128):
            pltpu.sync_copy(zrow16,
                            cnt_sh.at[pl.ds(tid * _RPT + j * 128, 128)])

    pltpu.sync_copy(edges.at[0, wid], sidx)
    pltpu.sync_copy(edges.at[1, wid], didx)

    plsc.subcore_barrier()

    def _dix(g):
        return didx.at[g]

    def _six(g):
        return sidx.at[g]

    def _drain_b(b):
        pltpu.make_async_copy(rows.at[b], agg_sh.at[_dix(0)], ssem[b]).wait()
        if with_count:
            pltpu.make_async_copy(ones, cnt_sh.at[_dix(0)], csem[b]).wait()

    def _group(gi, carry):
        base = gi * _NBUF
        for b in range(_NBUF):
            @pl.when(gi > 0)
            def _():
                _drain_b(b)

            pltpu.async_copy(feat.at[_six(base + b)], rows.at[b], gsem[b])
        for b in range(_NBUF):
            pltpu.make_async_copy(feat.at[_six(0)], rows.at[b],
                                  gsem[b]).wait()
            pltpu.async_copy(rows.at[b], agg_sh.at[_dix(base + b)],
                             ssem[b], add=True)
            if with_count:
                pltpu.async_copy(ones, cnt_sh.at[_dix(base + b)],
                                 csem[b], add=True)
        return carry

    lax.fori_loop(0, _NGRP, _group, 0)
    for b in range(_NBUF):
        _drain_b(b)

    plsc.subcore_barrier()

    pltpu.sync_copy(agg_sh.at[pl.ds(tid * _RPT, _RPT)],
                    agg_out.at[cid, pl.ds(tid * _RPT, _RPT)])
    if with_count:
        pltpu.sync_copy(cnt_sh.at[pl.ds(tid * _RPT, _RPT)],
                        cnt_out.at[cid, pl.ds(tid * _RPT, _RPT)])


_sc_params = pltpu.CompilerParams(use_tc_tiling_on_sc=False)

_seg_cnt = pl.kernel(
    functools.partial(_seg_body, True),
    compiler_params=_sc_params,
    out_type=[
        jax.ShapeDtypeStruct((_NC, _NPAD, _DH), jnp.float32),
        jax.ShapeDtypeStruct((_NC, _NPAD, _DH), jnp.int16),
    ],
    mesh=_mesh,
    scratch_types=[
        pltpu.VMEM_SHARED((_NPAD, _DH), jnp.float32),
        pltpu.VMEM_SHARED((_NPAD, _DH), jnp.int16),
        pltpu.VMEM((_NCH, _CHUNK), jnp.int32),
        pltpu.VMEM((_NCH, _CHUNK), jnp.int32),
        pltpu.VMEM((_NBUF, _CHUNK, _DH), jnp.float32),
        pltpu.VMEM((_CHUNK, _DH), jnp.int16),
        pltpu.VMEM((128, _DH), jnp.float32),
        pltpu.VMEM((128, _DH), jnp.int16),
    ] + [pltpu.SemaphoreType.DMA] * (3 * _NBUF),
)

_seg = pl.kernel(
    functools.partial(_seg_body, False),
    compiler_params=_sc_params,
    out_type=jax.ShapeDtypeStruct((_NC, _NPAD, _DH), jnp.float32),
    mesh=_mesh,
    scratch_types=[
        pltpu.VMEM_SHARED((_NPAD, _DH), jnp.float32),
        pltpu.VMEM((_NCH, _CHUNK), jnp.int32),
        pltpu.VMEM((_NCH, _CHUNK), jnp.int32),
        pltpu.VMEM((_NBUF, _CHUNK, _DH), jnp.float32),
        pltpu.VMEM((128, _DH), jnp.float32),
    ] + [pltpu.SemaphoreType.DMA] * (2 * _NBUF),
)

_BLKN = 2048
_BLKP = _BLKN * _DH // 128
_GRID = _NPAD // _BLKN


def _mm1_body(x_ref, w_ref, y_ref, z_ref, ty_ref, tz_ref):
    t = lax.dot_general(
        x_ref[...], w_ref[...], (((1,), (1,)), ((), ())),
        preferred_element_type=jnp.float32)
    ty_ref[...] = t[:, 0:_DH]
    tz_ref[...] = t[:, _DH:2 * _DH]
    for n8 in range(8):
        y_ref[:, n8 * _DH:(n8 + 1) * _DH] = ty_ref[n8::8, :]
        z_ref[:, n8 * _DH:(n8 + 1) * _DH] = tz_ref[n8::8, :]


_mm1 = pl.pallas_call(
    _mm1_body,
    grid=(_GRID,),
    in_specs=[
        pl.BlockSpec((_BLKN, _DIN), lambda i: (i, 0)),
        pl.BlockSpec((2 * _DH, _DIN), lambda i: (0, 0)),
    ],
    out_specs=[
        pl.BlockSpec((_BLKP, 128), lambda i: (i, 0)),
        pl.BlockSpec((_BLKP, 128), lambda i: (i, 0)),
    ],
    out_shape=[
        jax.ShapeDtypeStruct((_PK, 128), jnp.float32),
        jax.ShapeDtypeStruct((_PK, 128), jnp.float32),
    ],
    scratch_shapes=[pltpu.VMEM((_BLKN, _DH), jnp.float32),
                    pltpu.VMEM((_BLKN, _DH), jnp.float32)],
)


def _tc2_body(ap, cp, zp, b1p, h_ref, inv_ref):
    a = ap[0] + ap[1]
    c = cp[0].astype(jnp.float32) + cp[1].astype(jnp.float32)
    inv = 1.0 / jnp.maximum(c, 1.0)
    inv_ref[...] = inv
    h_ref[...] = jnp.maximum(a * inv + zp[...] + b1p[...], 0.0)


_tc2 = pl.pallas_call(
    _tc2_body,
    grid=(_GRID,),
    in_specs=[
        pl.BlockSpec((_NC, _BLKP, 128), lambda i: (0, i, 0)),
        pl.BlockSpec((_NC, _BLKP, 128), lambda i: (0, i, 0)),
        pl.BlockSpec((_BLKP, 128), lambda i: (i, 0)),
        pl.BlockSpec((1, 128), lambda i: (0, 0)),
    ],
    out_specs=[
        pl.BlockSpec((_BLKP, 128), lambda i: (i, 0)),
        pl.BlockSpec((_BLKP, 128), lambda i: (i, 0)),
    ],
    out_shape=[
        jax.ShapeDtypeStruct((_PK, 128), jnp.float32),
        jax.ShapeDtypeStruct((_PK, 128), jnp.float32),
    ],
)


def _tc3_body(gp, invp, hp, bm, bh, b2t, o_ref):
    m = (gp[0] + gp[1]) * invp[...]
    op = (
        lax.dot_general(m, bm[...], (((1,), (0,)), ((), ())),
                        preferred_element_type=jnp.float32)
        + lax.dot_general(hp[...], bh[...], (((1,), (0,)), ((), ())),
                          preferred_element_type=jnp.float32)
        + b2t[...])
    for n8 in range(8):
        o_ref[n8::8, :] = op[:, n8 * _DOUT:(n8 + 1) * _DOUT]


_tc3 = pl.pallas_call(
    _tc3_body,
    grid=(_GRID,),
    in_specs=[
        pl.BlockSpec((_NC, _BLKP, 128), lambda i: (0, i, 0)),
        pl.BlockSpec((_BLKP, 128), lambda i: (i, 0)),
        pl.BlockSpec((_BLKP, 128), lambda i: (i, 0)),
        pl.BlockSpec((8 * _DH, 8 * _DOUT), lambda i: (0, 0)),
        pl.BlockSpec((8 * _DH, 8 * _DOUT), lambda i: (0, 0)),
        pl.BlockSpec((1, 8 * _DOUT), lambda i: (0, 0)),
    ],
    out_specs=pl.BlockSpec((_BLKN, _DOUT), lambda i: (i, 0)),
    out_shape=jax.ShapeDtypeStruct((_N, _DOUT), jnp.float32),
)


def _expand_w2(w):
    eye8 = jnp.eye(8, dtype=w.dtype)
    return jnp.einsum("ab,fo->afbo", eye8, w.T).reshape(8 * _DH, 8 * _DOUT)


def kernel(x, edge_index, W1_l, b1, W1_r, W2_l, b2, W2_r):
    edges = edge_index.astype(jnp.int32).reshape(2, _NW, _NCH, _CHUNK)
    W1cat = jnp.concatenate([W1_l, W1_r], axis=0)
    y1p, z1p = _mm1(x, W1cat)
    aggp, cntp = _seg_cnt(y1p.reshape(_NPAD, _DH), edges)
    hp, invp = _tc2(aggp.reshape(_NC, _PK, 128),
                    cntp.reshape(_NC, _PK, 128),
                    z1p, jnp.tile(b1, 8).reshape(1, 128))
    gp = _seg(hp.reshape(_NPAD, _DH), edges)
    return _tc3(gp.reshape(_NC, _PK, 128), invp, hp,
                _expand_w2(W2_l), _expand_w2(W2_r),
                jnp.tile(b2, 8).reshape(1, 8 * _DOUT))


# speedup vs baseline: 38.6315x; 1.0562x over previous
import functools

import jax
import jax.numpy as jnp
from jax import lax
from jax.experimental import pallas as pl
from jax.experimental.pallas import tpu as pltpu
from jax.experimental.pallas import tpu_sc as plsc

_N = 10000
_E = 320000
_DIN = 128
_DH = 16
_DOUT = 2
_NC = 2
_NS = 16
_NW = _NC * _NS
_EW = _E // _NW
_CHUNK = 400
_NCH = _EW // _CHUNK
_NPAD = 10240
_RPT = _NPAD // _NS
_PK = _NPAD * _DH // 128

_mesh = plsc.VectorSubcoreMesh(
    core_axis_name="c", subcore_axis_name="s", num_cores=_NC, num_subcores=_NS
)

_NBUF = 5
_NGRP = _NCH // _NBUF


def _seg_body(feat, edges, agg_out, agg_sh, sidx, didx, rows, zrow, *sems):
    gsem = sems[0:_NBUF]
    ssem = sems[_NBUF:2 * _NBUF]
    cid = lax.axis_index("c")
    tid = lax.axis_index("s")
    wid = tid * _NC + cid

    z16 = jnp.zeros((16,), jnp.float32)

    def _zb(i, carry):
        zrow[i, :] = z16
        return carry

    lax.fori_loop(0, 128, _zb, 0)
    for j in range(_RPT // 128):
        pltpu.sync_copy(zrow, agg_sh.at[pl.ds(tid * _RPT + j * 128, 128)])

    pltpu.sync_copy(edges.at[0, wid], sidx)
    pltpu.sync_copy(edges.at[1, wid], didx)

    plsc.subcore_barrier()

    def _drain_b(b):
        pltpu.make_async_copy(rows.at[b], agg_sh.at[didx.at[0]],
                              ssem[b]).wait()

    def _group(gi, carry):
        base = gi * _NBUF
        for b in range(_NBUF):
            @pl.when(gi > 0)
            def _():
                _drain_b(b)

            pltpu.async_copy(feat.at[sidx.at[base + b]], rows.at[b], gsem[b])
        for b in range(_NBUF):
            pltpu.make_async_copy(feat.at[sidx.at[0]], rows.at[b],
                                  gsem[b]).wait()
            pltpu.async_copy(rows.at[b], agg_sh.at[didx.at[base + b]],
                             ssem[b], add=True)
        return carry

    lax.fori_loop(0, _NGRP, _group, 0)
    for b in range(_NBUF):
        _drain_b(b)

    plsc.subcore_barrier()

    pltpu.sync_copy(agg_sh.at[pl.ds(tid * _RPT, _RPT)],
                    agg_out.at[cid, pl.ds(tid * _RPT, _RPT)])


def _cnt_body(edges, cnt_out, cnt_sh, didx, ones, zrow16, *csem):
    cid = lax.axis_index("c")
    tid = lax.axis_index("s")
    wid = tid * _NC + cid

    o2 = jnp.ones((2, 16), jnp.int16)
    zs2 = jnp.zeros((2, 16), jnp.int16)

    def _ob(i, carry):
        ones[pl.ds(i * 2, 2), :] = o2
        return carry

    lax.fori_loop(0, _CHUNK // 2, _ob, 0)

    def _zsb(i, carry):
        zrow16[pl.ds(i * 2, 2), :] = zs2
        return carry

    lax.fori_loop(0, 64, _zsb, 0)
    for j in range(_RPT // 128):
        pltpu.sync_copy(zrow16, cnt_sh.at[pl.ds(tid * _RPT + j * 128, 128)])

    pltpu.sync_copy(edges.at[1, wid], didx)

    plsc.subcore_barrier()

    def _cgroup(gi, carry):
        base = gi * _NBUF
        for b in range(_NBUF):
            pltpu.async_copy(ones, cnt_sh.at[didx.at[base + b]],
                             csem[b], add=True)
        return carry

    lax.fori_loop(0, _NGRP, _cgroup, 0)
    for _ in range(_NGRP):
        for b in range(_NBUF):
            pltpu.make_async_copy(ones, cnt_sh.at[didx.at[0]],
                                  csem[b]).wait()

    plsc.subcore_barrier()

    pltpu.sync_copy(cnt_sh.at[pl.ds(tid * _RPT, _RPT)],
                    cnt_out.at[cid, pl.ds(tid * _RPT, _RPT)])


_sc_params = pltpu.CompilerParams(use_tc_tiling_on_sc=False)

_cnt = pl.kernel(
    _cnt_body,
    compiler_params=_sc_params,
    out_type=jax.ShapeDtypeStruct((_NC, _NPAD, _DH), jnp.int16),
    mesh=_mesh,
    scratch_types=[
        pltpu.VMEM_SHARED((_NPAD, _DH), jnp.int16),
        pltpu.VMEM((_NCH, _CHUNK), jnp.int32),
        pltpu.VMEM((_CHUNK, _DH), jnp.int16),
        pltpu.VMEM((128, _DH), jnp.int16),
    ] + [pltpu.SemaphoreType.DMA] * _NBUF,
)

_seg = pl.kernel(
    _seg_body,
    compiler_params=_sc_params,
    out_type=jax.ShapeDtypeStruct((_NC, _NPAD, _DH), jnp.float32),
    mesh=_mesh,
    scratch_types=[
        pltpu.VMEM_SHARED((_NPAD, _DH), jnp.float32),
        pltpu.VMEM((_NCH, _CHUNK), jnp.int32),
        pltpu.VMEM((_NCH, _CHUNK), jnp.int32),
        pltpu.VMEM((_NBUF, _CHUNK, _DH), jnp.float32),
        pltpu.VMEM((128, _DH), jnp.float32),
    ] + [pltpu.SemaphoreType.DMA] * (2 * _NBUF),
)

_BLKN = 2048
_BLKP = _BLKN * _DH // 128
_GRID = _NPAD // _BLKN


def _mm1_body(x_ref, w_ref, y_ref, z_ref, ty_ref, tz_ref):
    t = lax.dot_general(
        x_ref[...], w_ref[...], (((1,), (1,)), ((), ())),
        preferred_element_type=jnp.float32)
    ty_ref[...] = t[:, 0:_DH]
    tz_ref[...] = t[:, _DH:2 * _DH]
    for n8 in range(8):
        y_ref[:, n8 * _DH:(n8 + 1) * _DH] = ty_ref[n8::8, :]
        z_ref[:, n8 * _DH:(n8 + 1) * _DH] = tz_ref[n8::8, :]


_mm1 = pl.pallas_call(
    _mm1_body,
    grid=(_GRID,),
    in_specs=[
        pl.BlockSpec((_BLKN, _DIN), lambda i: (i, 0)),
        pl.BlockSpec((2 * _DH, _DIN), lambda i: (0, 0)),
    ],
    out_specs=[
        pl.BlockSpec((_BLKP, 128), lambda i: (i, 0)),
        pl.BlockSpec((_BLKP, 128), lambda i: (i, 0)),
    ],
    out_shape=[
        jax.ShapeDtypeStruct((_PK, 128), jnp.float32),
        jax.ShapeDtypeStruct((_PK, 128), jnp.float32),
    ],
    scratch_shapes=[pltpu.VMEM((_BLKN, _DH), jnp.float32),
                    pltpu.VMEM((_BLKN, _DH), jnp.float32)],
)


def _tc2_body(ap, cp, zp, b1p, h_ref, inv_ref):
    a = ap[0] + ap[1]
    c = cp[0].astype(jnp.float32) + cp[1].astype(jnp.float32)
    inv = 1.0 / jnp.maximum(c, 1.0)
    inv_ref[...] = inv
    h_ref[...] = jnp.maximum(a * inv + zp[...] + b1p[...], 0.0)


_tc2 = pl.pallas_call(
    _tc2_body,
    grid=(_GRID,),
    in_specs=[
        pl.BlockSpec((_NC, _BLKP, 128), lambda i: (0, i, 0)),
        pl.BlockSpec((_NC, _BLKP, 128), lambda i: (0, i, 0)),
        pl.BlockSpec((_BLKP, 128), lambda i: (i, 0)),
        pl.BlockSpec((1, 128), lambda i: (0, 0)),
    ],
    out_specs=[
        pl.BlockSpec((_BLKP, 128), lambda i: (i, 0)),
        pl.BlockSpec((_BLKP, 128), lambda i: (i, 0)),
    ],
    out_shape=[
        jax.ShapeDtypeStruct((_PK, 128), jnp.float32),
        jax.ShapeDtypeStruct((_PK, 128), jnp.float32),
    ],
)


def _tc3_body(gp, invp, hp, bm, bh, b2t, o_ref):
    m = (gp[0] + gp[1]) * invp[...]
    op = (
        lax.dot_general(m, bm[...], (((1,), (0,)), ((), ())),
                        preferred_element_type=jnp.float32)
        + lax.dot_general(hp[...], bh[...], (((1,), (0,)), ((), ())),
                          preferred_element_type=jnp.float32)
        + b2t[...])
    for n8 in range(8):
        o_ref[n8::8, :] = op[:, n8 * _DOUT:(n8 + 1) * _DOUT]


_tc3 = pl.pallas_call(
    _tc3_body,
    grid=(_GRID,),
    in_specs=[
        pl.BlockSpec((_NC, _BLKP, 128), lambda i: (0, i, 0)),
        pl.BlockSpec((_BLKP, 128), lambda i: (i, 0)),
        pl.BlockSpec((_BLKP, 128), lambda i: (i, 0)),
        pl.BlockSpec((8 * _DH, 8 * _DOUT), lambda i: (0, 0)),
        pl.BlockSpec((8 * _DH, 8 * _DOUT), lambda i: (0, 0)),
        pl.BlockSpec((1, 8 * _DOUT), lambda i: (0, 0)),
    ],
    out_specs=pl.BlockSpec((_BLKN, _DOUT), lambda i: (i, 0)),
    out_shape=jax.ShapeDtypeStruct((_N, _DOUT), jnp.float32),
)


def _expand_w2(w):
    eye8 = jnp.eye(8, dtype=w.dtype)
    return jnp.einsum("ab,fo->afbo", eye8, w.T).reshape(8 * _DH, 8 * _DOUT)


def kernel(x, edge_index, W1_l, b1, W1_r, W2_l, b2, W2_r):
    edges = edge_index.astype(jnp.int32).reshape(2, _NW, _NCH, _CHUNK)
    W1cat = jnp.concatenate([W1_l, W1_r], axis=0)
    cntp = _cnt(edges)
    y1p, z1p = _mm1(x, W1cat)
    aggp = _seg(y1p.reshape(_NPAD, _DH), edges)
    hp, invp = _tc2(aggp.reshape(_NC, _PK, 128),
                    cntp.reshape(_NC, _PK, 128),
                    z1p, jnp.tile(b1, 8).reshape(1, 128))
    gp = _seg(hp.reshape(_NPAD, _DH), edges)
    return _tc3(gp.reshape(_NC, _PK, 128), invp, hp,
                _expand_w2(W2_l), _expand_w2(W2_r),
                jnp.tile(b2, 8).reshape(1, 8 * _DOUT))


# speedup vs baseline: 39.1241x; 1.0128x over previous
import functools

import jax
import jax.numpy as jnp
from jax import lax
from jax.experimental import pallas as pl
from jax.experimental.pallas import tpu as pltpu
from jax.experimental.pallas import tpu_sc as plsc

_N = 10000
_E = 320000
_DIN = 128
_DH = 16
_DOUT = 2
_NC = 2
_NS = 16
_NW = _NC * _NS
_EW = _E // _NW
_CHUNK = 200
_NCH = _EW // _CHUNK
_NPAD = 10240
_RPT = _NPAD // _NS
_PK = _NPAD * _DH // 128

_mesh = plsc.VectorSubcoreMesh(
    core_axis_name="c", subcore_axis_name="s", num_cores=_NC, num_subcores=_NS
)

_NBUF = 10
_NGRP = _NCH // _NBUF


def _seg_body(feat, edges, agg_out, agg_sh, sidx, didx, rows, zrow, *sems):
    gsem = sems[0:_NBUF]
    ssem = sems[_NBUF:2 * _NBUF]
    cid = lax.axis_index("c")
    tid = lax.axis_index("s")
    wid = tid * _NC + cid

    z16 = jnp.zeros((16,), jnp.float32)

    def _zb(i, carry):
        zrow[i, :] = z16
        return carry

    lax.fori_loop(0, 128, _zb, 0)
    for j in range(_RPT // 128):
        pltpu.sync_copy(zrow, agg_sh.at[pl.ds(tid * _RPT + j * 128, 128)])

    pltpu.sync_copy(edges.at[0, wid], sidx)
    pltpu.sync_copy(edges.at[1, wid], didx)

    plsc.subcore_barrier()

    def _drain_b(b):
        pltpu.make_async_copy(rows.at[b], agg_sh.at[didx.at[0]],
                              ssem[b]).wait()

    def _group(gi, carry):
        base = gi * _NBUF
        for b in range(_NBUF):
            @pl.when(gi > 0)
            def _():
                _drain_b(b)

            pltpu.async_copy(feat.at[sidx.at[base + b]], rows.at[b], gsem[b])
        for b in range(_NBUF):
            pltpu.make_async_copy(feat.at[sidx.at[0]], rows.at[b],
                                  gsem[b]).wait()
            pltpu.async_copy(rows.at[b], agg_sh.at[didx.at[base + b]],
                             ssem[b], add=True)
        return carry

    lax.fori_loop(0, _NGRP, _group, 0)
    for b in range(_NBUF):
        _drain_b(b)

    plsc.subcore_barrier()

    pltpu.sync_copy(agg_sh.at[pl.ds(tid * _RPT, _RPT)],
                    agg_out.at[cid, pl.ds(tid * _RPT, _RPT)])


def _cnt_body(edges, cnt_out, cnt_sh, didx, ones, zrow16, *csem):
    cid = lax.axis_index("c")
    tid = lax.axis_index("s")
    wid = tid * _NC + cid

    o2 = jnp.ones((2, 16), jnp.int16)
    zs2 = jnp.zeros((2, 16), jnp.int16)

    def _ob(i, carry):
        ones[pl.ds(i * 2, 2), :] = o2
        return carry

    lax.fori_loop(0, _CHUNK // 2, _ob, 0)

    def _zsb(i, carry):
        zrow16[pl.ds(i * 2, 2), :] = zs2
        return carry

    lax.fori_loop(0, 64, _zsb, 0)
    for j in range(_RPT // 128):
        pltpu.sync_copy(zrow16, cnt_sh.at[pl.ds(tid * _RPT + j * 128, 128)])

    pltpu.sync_copy(edges.at[1, wid], didx)

    plsc.subcore_barrier()

    def _cgroup(gi, carry):
        base = gi * _NBUF
        for b in range(_NBUF):
            pltpu.async_copy(ones, cnt_sh.at[didx.at[base + b]],
                             csem[b], add=True)
        return carry

    lax.fori_loop(0, _NGRP, _cgroup, 0)
    for _ in range(_NGRP):
        for b in range(_NBUF):
            pltpu.make_async_copy(ones, cnt_sh.at[didx.at[0]],
                                  csem[b]).wait()

    plsc.subcore_barrier()

    pltpu.sync_copy(cnt_sh.at[pl.ds(tid * _RPT, _RPT)],
                    cnt_out.at[cid, pl.ds(tid * _RPT, _RPT)])


_sc_params = pltpu.CompilerParams(use_tc_tiling_on_sc=False)

_cnt = pl.kernel(
    _cnt_body,
    compiler_params=_sc_params,
    out_type=jax.ShapeDtypeStruct((_NC, _NPAD, _DH), jnp.int16),
    mesh=_mesh,
    scratch_types=[
        pltpu.VMEM_SHARED((_NPAD, _DH), jnp.int16),
        pltpu.VMEM((_NCH, _CHUNK), jnp.int32),
        pltpu.VMEM((_CHUNK, _DH), jnp.int16),
        pltpu.VMEM((128, _DH), jnp.int16),
    ] + [pltpu.SemaphoreType.DMA] * _NBUF,
)

_seg = pl.kernel(
    _seg_body,
    compiler_params=_sc_params,
    out_type=jax.ShapeDtypeStruct((_NC, _NPAD, _DH), jnp.float32),
    mesh=_mesh,
    scratch_types=[
        pltpu.VMEM_SHARED((_NPAD, _DH), jnp.float32),
        pltpu.VMEM((_NCH, _CHUNK), jnp.int32),
        pltpu.VMEM((_NCH, _CHUNK), jnp.int32),
        pltpu.VMEM((_NBUF, _CHUNK, _DH), jnp.float32),
        pltpu.VMEM((128, _DH), jnp.float32),
    ] + [pltpu.SemaphoreType.DMA] * (2 * _NBUF),
)

_BLKN = 2048
_BLKP = _BLKN * _DH // 128
_GRID = _NPAD // _BLKN


def _mm1_body(x_ref, w_ref, y_ref, z_ref, ty_ref, tz_ref):
    t = lax.dot_general(
        x_ref[...], w_ref[...], (((1,), (1,)), ((), ())),
        preferred_element_type=jnp.float32)
    ty_ref[...] = t[:, 0:_DH]
    tz_ref[...] = t[:, _DH:2 * _DH]
    for n8 in range(8):
        y_ref[:, n8 * _DH:(n8 + 1) * _DH] = ty_ref[n8::8, :]
        z_ref[:, n8 * _DH:(n8 + 1) * _DH] = tz_ref[n8::8, :]


_mm1 = pl.pallas_call(
    _mm1_body,
    grid=(_GRID,),
    in_specs=[
        pl.BlockSpec((_BLKN, _DIN), lambda i: (i, 0)),
        pl.BlockSpec((2 * _DH, _DIN), lambda i: (0, 0)),
    ],
    out_specs=[
        pl.BlockSpec((_BLKP, 128), lambda i: (i, 0)),
        pl.BlockSpec((_BLKP, 128), lambda i: (i, 0)),
    ],
    out_shape=[
        jax.ShapeDtypeStruct((_PK, 128), jnp.float32),
        jax.ShapeDtypeStruct((_PK, 128), jnp.float32),
    ],
    scratch_shapes=[pltpu.VMEM((_BLKN, _DH), jnp.float32),
                    pltpu.VMEM((_BLKN, _DH), jnp.float32)],
)


def _tc2_body(ap, cp, zp, b1p, h_ref, inv_ref):
    a = ap[0] + ap[1]
    c = cp[0].astype(jnp.float32) + cp[1].astype(jnp.float32)
    inv = 1.0 / jnp.maximum(c, 1.0)
    inv_ref[...] = inv
    h_ref[...] = jnp.maximum(a * inv + zp[...] + b1p[...], 0.0)


_tc2 = pl.pallas_call(
    _tc2_body,
    grid=(_GRID,),
    in_specs=[
        pl.BlockSpec((_NC, _BLKP, 128), lambda i: (0, i, 0)),
        pl.BlockSpec((_NC, _BLKP, 128), lambda i: (0, i, 0)),
        pl.BlockSpec((_BLKP, 128), lambda i: (i, 0)),
        pl.BlockSpec((1, 128), lambda i: (0, 0)),
    ],
    out_specs=[
        pl.BlockSpec((_BLKP, 128), lambda i: (i, 0)),
        pl.BlockSpec((_BLKP, 128), lambda i: (i, 0)),
    ],
    out_shape=[
        jax.ShapeDtypeStruct((_PK, 128), jnp.float32),
        jax.ShapeDtypeStruct((_PK, 128), jnp.float32),
    ],
)


def _tc3_body(gp, invp, hp, bm, bh, b2t, o_ref):
    m = (gp[0] + gp[1]) * invp[...]
    op = (
        lax.dot_general(m, bm[...], (((1,), (0,)), ((), ())),
                        preferred_element_type=jnp.float32)
        + lax.dot_general(hp[...], bh[...], (((1,), (0,)), ((), ())),
                          preferred_element_type=jnp.float32)
        + b2t[...])
    for n8 in range(8):
        o_ref[n8::8, :] = op[:, n8 * _DOUT:(n8 + 1) * _DOUT]


_tc3 = pl.pallas_call(
    _tc3_body,
    grid=(_GRID,),
    in_specs=[
        pl.BlockSpec((_NC, _BLKP, 128), lambda i: (0, i, 0)),
        pl.BlockSpec((_BLKP, 128), lambda i: (i, 0)),
        pl.BlockSpec((_BLKP, 128), lambda i: (i, 0)),
        pl.BlockSpec((8 * _DH, 8 * _DOUT), lambda i: (0, 0)),
        pl.BlockSpec((8 * _DH, 8 * _DOUT), lambda i: (0, 0)),
        pl.BlockSpec((1, 8 * _DOUT), lambda i: (0, 0)),
    ],
    out_specs=pl.BlockSpec((_BLKN, _DOUT), lambda i: (i, 0)),
    out_shape=jax.ShapeDtypeStruct((_N, _DOUT), jnp.float32),
)


def _expand_w2(w):
    eye8 = jnp.eye(8, dtype=w.dtype)
    return jnp.einsum("ab,fo->afbo", eye8, w.T).reshape(8 * _DH, 8 * _DOUT)


def kernel(x, edge_index, W1_l, b1, W1_r, W2_l, b2, W2_r):
    edges = edge_index.astype(jnp.int32).reshape(2, _NW, _NCH, _CHUNK)
    W1cat = jnp.concatenate([W1_l, W1_r], axis=0)
    cntp = _cnt(edges)
    y1p, z1p = _mm1(x, W1cat)
    aggp = _seg(y1p.reshape(_NPAD, _DH), edges)
    hp, invp = _tc2(aggp.reshape(_NC, _PK, 128),
                    cntp.reshape(_NC, _PK, 128),
                    z1p, jnp.tile(b1, 8).reshape(1, 128))
    gp = _seg(hp.reshape(_NPAD, _DH), edges)
    return _tc3(gp.reshape(_NC, _PK, 128), invp, hp,
                _expand_w2(W2_l), _expand_w2(W2_r),
                jnp.tile(b2, 8).reshape(1, 8 * _DOUT))


# speedup vs baseline: 40.1194x; 1.0254x over previous
import functools

import jax
import jax.numpy as jnp
from jax import lax
from jax.experimental import pallas as pl
from jax.experimental.pallas import tpu as pltpu
from jax.experimental.pallas import tpu_sc as plsc

_N = 10000
_E = 320000
_DIN = 128
_DH = 16
_DOUT = 2
_NC = 2
_NS = 16
_NW = _NC * _NS
_EW = _E // _NW
_CHUNK = 200
_NCH = _EW // _CHUNK
_NPAD = 10240
_RPT = _NPAD // _NS
_PK = _NPAD * _DH // 128

_mesh = plsc.VectorSubcoreMesh(
    core_axis_name="c", subcore_axis_name="s", num_cores=_NC, num_subcores=_NS
)

_NBUF = 10
_NGRP = _NCH // _NBUF


def _seg_body(feat, edges, agg_out, agg_sh, sidx, didx, rows, zrow, *sems):
    gsem = sems[0:_NBUF]
    ssem = sems[_NBUF:2 * _NBUF]
    cid = lax.axis_index("c")
    tid = lax.axis_index("s")
    wid = tid * _NC + cid

    z16 = jnp.zeros((16,), jnp.float32)

    def _zb(i, carry):
        zrow[i, :] = z16
        return carry

    lax.fori_loop(0, 128, _zb, 0)
    for j in range(_RPT // 128):
        pltpu.sync_copy(zrow, agg_sh.at[pl.ds(tid * _RPT + j * 128, 128)])

    pltpu.sync_copy(edges.at[0, wid], sidx)
    pltpu.sync_copy(edges.at[1, wid], didx)

    plsc.subcore_barrier()

    def _drain_b(b):
        pltpu.make_async_copy(rows.at[b], agg_sh.at[didx.at[0]],
                              ssem[b]).wait()

    def _group(gi, carry):
        base = gi * _NBUF
        for b in range(_NBUF):
            @pl.when(gi > 0)
            def _():
                _drain_b(b)

            pltpu.async_copy(feat.at[sidx.at[base + b]], rows.at[b], gsem[b])
        for b in range(_NBUF):
            pltpu.make_async_copy(feat.at[sidx.at[0]], rows.at[b],
                                  gsem[b]).wait()
            pltpu.async_copy(rows.at[b], agg_sh.at[didx.at[base + b]],
                             ssem[b], add=True)
        return carry

    lax.fori_loop(0, _NGRP, _group, 0)
    for b in range(_NBUF):
        _drain_b(b)

    plsc.subcore_barrier()

    pltpu.sync_copy(agg_sh.at[pl.ds(tid * _RPT, _RPT)],
                    agg_out.at[cid, pl.ds(tid * _RPT, _RPT)])


def _cnt_body(edges, cnt_out, cnt_sh, didx, ones, zrow16, *csem):
    cid = lax.axis_index("c")
    tid = lax.axis_index("s")
    wid = tid * _NC + cid

    o2 = jnp.ones((2, 16), jnp.int16)
    zs2 = jnp.zeros((2, 16), jnp.int16)

    def _ob(i, carry):
        ones[pl.ds(i * 2, 2), :] = o2
        return carry

    lax.fori_loop(0, _CHUNK // 2, _ob, 0)

    def _zsb(i, carry):
        zrow16[pl.ds(i * 2, 2), :] = zs2
        return carry

    lax.fori_loop(0, 64, _zsb, 0)
    for j in range(_RPT // 128):
        pltpu.sync_copy(zrow16, cnt_sh.at[pl.ds(tid * _RPT + j * 128, 128)])

    pltpu.sync_copy(edges.at[1, wid], didx)

    plsc.subcore_barrier()

    def _cgroup(gi, carry):
        base = gi * _NBUF
        for b in range(_NBUF):
            pltpu.async_copy(ones, cnt_sh.at[didx.at[base + b]],
                             csem[b], add=True)
        return carry

    lax.fori_loop(0, _NGRP, _cgroup, 0)
    for _ in range(_NGRP):
        for b in range(_NBUF):
            pltpu.make_async_copy(ones, cnt_sh.at[didx.at[0]],
                                  csem[b]).wait()

    plsc.subcore_barrier()

    pltpu.sync_copy(cnt_sh.at[pl.ds(tid * _RPT, _RPT)],
                    cnt_out.at[cid, pl.ds(tid * _RPT, _RPT)])


_sc_params = pltpu.CompilerParams(use_tc_tiling_on_sc=False)

_cnt = pl.kernel(
    _cnt_body,
    compiler_params=_sc_params,
    out_type=jax.ShapeDtypeStruct((_NC, _NPAD, _DH), jnp.int16),
    mesh=_mesh,
    scratch_types=[
        pltpu.VMEM_SHARED((_NPAD, _DH), jnp.int16),
        pltpu.VMEM((_NCH, _CHUNK), jnp.int32),
        pltpu.VMEM((_CHUNK, _DH), jnp.int16),
        pltpu.VMEM((128, _DH), jnp.int16),
    ] + [pltpu.SemaphoreType.DMA] * _NBUF,
)

_seg = pl.kernel(
    _seg_body,
    compiler_params=_sc_params,
    out_type=jax.ShapeDtypeStruct((_NC, _NPAD, _DH), jnp.float32),
    mesh=_mesh,
    scratch_types=[
        pltpu.VMEM_SHARED((_NPAD, _DH), jnp.float32),
        pltpu.VMEM((_NCH, _CHUNK), jnp.int32),
        pltpu.VMEM((_NCH, _CHUNK), jnp.int32),
        pltpu.VMEM((_NBUF, _CHUNK, _DH), jnp.float32),
        pltpu.VMEM((128, _DH), jnp.float32),
    ] + [pltpu.SemaphoreType.DMA] * (2 * _NBUF),
)

_BLKN = 10240
_BLKP = _BLKN * _DH // 128
_GRID = _NPAD // _BLKN


def _mm1_body(x_ref, w_ref, y_ref, z_ref, ty_ref, tz_ref):
    t = lax.dot_general(
        x_ref[...], w_ref[...], (((1,), (1,)), ((), ())),
        preferred_element_type=jnp.float32)
    ty_ref[...] = t[:, 0:_DH]
    tz_ref[...] = t[:, _DH:2 * _DH]
    for n8 in range(8):
        y_ref[:, n8 * _DH:(n8 + 1) * _DH] = ty_ref[n8::8, :]
        z_ref[:, n8 * _DH:(n8 + 1) * _DH] = tz_ref[n8::8, :]


_mm1 = pl.pallas_call(
    _mm1_body,
    grid=(_GRID,),
    in_specs=[
        pl.BlockSpec((_BLKN, _DIN), lambda i: (i, 0)),
        pl.BlockSpec((2 * _DH, _DIN), lambda i: (0, 0)),
    ],
    out_specs=[
        pl.BlockSpec((_BLKP, 128), lambda i: (i, 0)),
        pl.BlockSpec((_BLKP, 128), lambda i: (i, 0)),
    ],
    out_shape=[
        jax.ShapeDtypeStruct((_PK, 128), jnp.float32),
        jax.ShapeDtypeStruct((_PK, 128), jnp.float32),
    ],
    scratch_shapes=[pltpu.VMEM((_BLKN, _DH), jnp.float32),
                    pltpu.VMEM((_BLKN, _DH), jnp.float32)],
)


def _tc2_body(ap, cp, zp, b1p, h_ref, inv_ref):
    a = ap[0] + ap[1]
    c = cp[0].astype(jnp.float32) + cp[1].astype(jnp.float32)
    inv = 1.0 / jnp.maximum(c, 1.0)
    inv_ref[...] = inv
    h_ref[...] = jnp.maximum(a * inv + zp[...] + b1p[...], 0.0)


_tc2 = pl.pallas_call(
    _tc2_body,
    grid=(_GRID,),
    in_specs=[
        pl.BlockSpec((_NC, _BLKP, 128), lambda i: (0, i, 0)),
        pl.BlockSpec((_NC, _BLKP, 128), lambda i: (0, i, 0)),
        pl.BlockSpec((_BLKP, 128), lambda i: (i, 0)),
        pl.BlockSpec((1, 128), lambda i: (0, 0)),
    ],
    out_specs=[
        pl.BlockSpec((_BLKP, 128), lambda i: (i, 0)),
        pl.BlockSpec((_BLKP, 128), lambda i: (i, 0)),
    ],
    out_shape=[
        jax.ShapeDtypeStruct((_PK, 128), jnp.float32),
        jax.ShapeDtypeStruct((_PK, 128), jnp.float32),
    ],
)


def _tc3_body(gp, invp, hp, bm, bh, b2t, o_ref):
    m = (gp[0] + gp[1]) * invp[...]
    op = (
        lax.dot_general(m, bm[...], (((1,), (0,)), ((), ())),
                        preferred_element_type=jnp.float32)
        + lax.dot_general(hp[...], bh[...], (((1,), (0,)), ((), ())),
                          preferred_element_type=jnp.float32)
        + b2t[...])
    for n8 in range(8):
        o_ref[n8::8, :] = op[:, n8 * _DOUT:(n8 + 1) * _DOUT]


_tc3 = pl.pallas_call(
    _tc3_body,
    grid=(_GRID,),
    in_specs=[
        pl.BlockSpec((_NC, _BLKP, 128), lambda i: (0, i, 0)),
        pl.BlockSpec((_BLKP, 128), lambda i: (i, 0)),
        pl.BlockSpec((_BLKP, 128), lambda i: (i, 0)),
        pl.BlockSpec((8 * _DH, 8 * _DOUT), lambda i: (0, 0)),
        pl.BlockSpec((8 * _DH, 8 * _DOUT), lambda i: (0, 0)),
        pl.BlockSpec((1, 8 * _DOUT), lambda i: (0, 0)),
    ],
    out_specs=pl.BlockSpec((_BLKN, _DOUT), lambda i: (i, 0)),
    out_shape=jax.ShapeDtypeStruct((_N, _DOUT), jnp.float32),
)


def _expand_w2(w):
    eye8 = jnp.eye(8, dtype=w.dtype)
    return jnp.einsum("ab,fo->afbo", eye8, w.T).reshape(8 * _DH, 8 * _DOUT)


def kernel(x, edge_index, W1_l, b1, W1_r, W2_l, b2, W2_r):
    edges = edge_index.astype(jnp.int32).reshape(2, _NW, _NCH, _CHUNK)
    W1cat = jnp.concatenate([W1_l, W1_r], axis=0)
    cntp = _cnt(edges)
    y1p, z1p = _mm1(x, W1cat)
    aggp = _seg(y1p.reshape(_NPAD, _DH), edges)
    hp, invp = _tc2(aggp.reshape(_NC, _PK, 128),
                    cntp.reshape(_NC, _PK, 128),
                    z1p, jnp.tile(b1, 8).reshape(1, 128))
    gp = _seg(hp.reshape(_NPAD, _DH), edges)
    return _tc3(gp.reshape(_NC, _PK, 128), invp, hp,
                _expand_w2(W2_l), _expand_w2(W2_r),
                jnp.tile(b2, 8).reshape(1, 8 * _DOUT))
